# baseline (speedup 1.0000x reference)
import functools

import jax
import jax.numpy as jnp
from jax import lax
from jax.experimental import pallas as pl
from jax.experimental.pallas import tpu as pltpu
from jax.experimental.pallas import tpu_sc as plsc

_L = 16


@functools.lru_cache(maxsize=None)
def _build(B, V, H):
    info = plsc.get_sparse_core_info()
    NC, NS = info.num_cores, info.num_subcores
    NW = NC * NS
    b_per_w = B // NW
    CH = 128
    n_ch = b_per_w // CH
    n_blk = b_per_w // _L

    mesh = plsc.VectorSubcoreMesh(core_axis_name="c", subcore_axis_name="s")

    @functools.partial(
        pl.kernel,
        mesh=mesh,
        out_type=jax.ShapeDtypeStruct((B, H), jnp.float32),
        scratch_types=[
            pltpu.VMEM((n_ch, CH), jnp.int32),
            pltpu.VMEM((b_per_w, H), jnp.float32),
            pltpu.VMEM((b_per_w, H), jnp.float32),
            pltpu.SemaphoreType.DMA,
        ],
        compiler_params=pltpu.CompilerParams(use_tc_tiling_on_sc=False),
    )
    def _k(ids_hbm, table_hbm, out_hbm, idx_v, rows_v, out_v, sem):
        wid = lax.axis_index("s") * NC + lax.axis_index("c")
        base = wid * b_per_w

        pltpu.sync_copy(ids_hbm.at[wid], idx_v)
        copies = [
            pltpu.async_copy(
                table_hbm.at[idx_v.at[j]],
                rows_v.at[pl.ds(j * CH, CH)],
                sem,
            )
            for j in range(n_ch)
        ]
        for c in copies:
            c.wait()

        lane = lax.iota(jnp.int32, _L)
        perms = [lane ^ s for s in (1, 2, 4, 8)]

        def row(i, carry):
            v = rows_v[i]
            m = v
            for p in perms:
                m = jnp.maximum(m, m.at[p].get(mode="promise_in_bounds"))
            e = jnp.exp(v - m)
            s = e
            for p in perms:
                s = s + s.at[p].get(mode="promise_in_bounds")
            out_v[i] = e / s
            return carry

        lax.fori_loop(0, b_per_w, row, None, unroll=8)

        pltpu.sync_copy(out_v, out_hbm.at[pl.ds(base, b_per_w)])

    return _k


def kernel(user_ids, logits_weight):
    B = user_ids.shape[0]
    V, H = logits_weight.shape
    info = plsc.get_sparse_core_info()
    NW = info.num_cores * info.num_subcores
    CH = 128
    ids = user_ids.astype(jnp.int32).reshape(NW, (B // NW) // CH, CH)
    return _build(B, V, H)(ids, logits_weight)


# speedup vs baseline: 1.7003x; 1.7003x over previous
import functools

import jax
import jax.numpy as jnp
from jax import lax
from jax.experimental import pallas as pl
from jax.experimental.pallas import tpu as pltpu
from jax.experimental.pallas import tpu_sc as plsc

_L = 16


@functools.lru_cache(maxsize=None)
def _build(B, V, H):
    info = plsc.get_sparse_core_info()
    NC, NS = info.num_cores, info.num_subcores
    NW = NC * NS
    b_per_w = B // NW

    mesh = plsc.VectorSubcoreMesh(core_axis_name="c", subcore_axis_name="s")

    @functools.partial(
        pl.kernel,
        mesh=mesh,
        out_type=jax.ShapeDtypeStruct((B, H), jnp.float32),
        scratch_types=[
            pltpu.VMEM((b_per_w,), jnp.int32),
            pltpu.VMEM((b_per_w, H), jnp.float32),
            pltpu.SemaphoreType.DMA,
        ],
    )
    def _k(ids_hbm, table_hbm, out_hbm, idx_v, rows_v, sem):
        wid = lax.axis_index("s") * NC + lax.axis_index("c")
        base = wid * b_per_w

        pltpu.sync_copy(ids_hbm.at[pl.ds(base, b_per_w)], idx_v)

        def fetch(j, carry):
            vec = idx_v[pl.ds(j * _L, _L)]
            for k in range(_L):
                uid = vec[k]
                pltpu.async_copy(table_hbm.at[uid], rows_v.at[j * _L + k], sem)
            return carry

        lax.fori_loop(0, b_per_w // _L, fetch, None, unroll=2)

        def drain(i, carry):
            pltpu.make_async_copy(table_hbm.at[0], rows_v.at[i], sem).wait()
            return carry

        lax.fori_loop(0, b_per_w, drain, None, unroll=8)

        lane = lax.iota(jnp.int32, _L)
        perms = [lane ^ s for s in (1, 2, 4, 8)]

        def row(i, carry):
            v = rows_v[i]
            m = v
            for p in perms:
                m = jnp.maximum(m, m.at[p].get(mode="promise_in_bounds"))
            e = jnp.exp(v - m)
            s = e
            for p in perms:
                s = s + s.at[p].get(mode="promise_in_bounds")
            rows_v[i] = e / s
            return carry

        lax.fori_loop(0, b_per_w, row, None, unroll=8)

        pltpu.sync_copy(rows_v, out_hbm.at[pl.ds(base, b_per_w)])

    return _k


def kernel(user_ids, logits_weight):
    B = user_ids.shape[0]
    V, H = logits_weight.shape
    return _build(B, V, H)(user_ids.astype(jnp.int32), logits_weight)


# speedup vs baseline: 1.7016x; 1.0008x over previous
import functools

import jax
import jax.numpy as jnp
from jax import lax
from jax.experimental import pallas as pl
from jax.experimental.pallas import tpu as pltpu
from jax.experimental.pallas import tpu_sc as plsc

_L = 16


@functools.lru_cache(maxsize=None)
def _build(B, V, H):
    info = plsc.get_sparse_core_info()
    NC, NS = info.num_cores, info.num_subcores
    NW = NC * NS
    b_per_w = B // NW

    mesh = plsc.VectorSubcoreMesh(core_axis_name="c", subcore_axis_name="s")

    @functools.partial(
        pl.kernel,
        mesh=mesh,
        out_type=jax.ShapeDtypeStruct((B, H), jnp.float32),
        scratch_types=[
            pltpu.VMEM((b_per_w,), jnp.int32),
            pltpu.VMEM((b_per_w, H), jnp.float32),
            pltpu.SemaphoreType.DMA,
        ],
    )
    def _k(ids_hbm, table_hbm, out_hbm, idx_v, rows_v, sem):
        wid = lax.axis_index("s") * NC + lax.axis_index("c")
        base = wid * b_per_w

        with jax.named_scope("phase_idx"):
            pltpu.sync_copy(ids_hbm.at[pl.ds(base, b_per_w)], idx_v)

        def fetch(j, carry):
            vec = idx_v[pl.ds(j * _L, _L)]
            for k in range(_L):
                uid = vec[k]
                pltpu.async_copy(table_hbm.at[uid], rows_v.at[j * _L + k], sem)
            return carry

        with jax.named_scope("phase_fetch"):
            lax.fori_loop(0, b_per_w // _L, fetch, None, unroll=2)

        def drain(i, carry):
            pltpu.make_async_copy(table_hbm.at[0], rows_v.at[i], sem).wait()
            return carry

        with jax.named_scope("phase_drain"):
            lax.fori_loop(0, b_per_w, drain, None, unroll=8)

        lane = lax.iota(jnp.int32, _L)
        perms = [lane ^ s for s in (1, 2, 4, 8)]

        def row(i, carry):
            v = rows_v[i]
            m = v
            for p in perms:
                m = jnp.maximum(m, m.at[p].get(mode="promise_in_bounds"))
            e = jnp.exp(v - m)
            s = e
            for p in perms:
                s = s + s.at[p].get(mode="promise_in_bounds")
            rows_v[i] = e / s
            return carry

        with jax.named_scope("phase_softmax"):
            lax.fori_loop(0, b_per_w, row, None, unroll=8)

        with jax.named_scope("phase_out"):
            pltpu.sync_copy(rows_v, out_hbm.at[pl.ds(base, b_per_w)])

    return _k


def kernel(user_ids, logits_weight):
    B = user_ids.shape[0]
    V, H = logits_weight.shape
    return _build(B, V, H)(user_ids.astype(jnp.int32), logits_weight)
